# baseline (speedup 1.0000x reference)
import functools

import jax
import jax.numpy as jnp
from jax import lax
from jax.experimental import pallas as pl
from jax.experimental.pallas import tpu as pltpu
from jax.experimental.pallas import tpu_sc as plsc

SEQ = 577
NUM_HEADS = 12
NB_R = 16
R_BLK = 40
ROWS_PAD = NB_R * R_BLK
SP = 592
NRD_PAD = 2304
HEAD_HALVES = 2
HEADS_PER_HALF = NUM_HEADS // HEAD_HALVES
SLAB = R_BLK * SP


def _sc_gather_body(table_hbm, idx_hbm, pos_hbm,
                    table_v, idx_v, out_v0, out_v1, tsem, isem, osem0, osem1):
    out_bufs = (out_v0, out_v1)
    osems = (osem0, osem1)
    wid = lax.axis_index("s") * 2 + lax.axis_index("c")
    hh = wid // NB_R
    rb = wid % NB_R
    h0 = hh * HEADS_PER_HALF

    tcopy = pltpu.make_async_copy(
        table_hbm.at[pl.ds(h0 * NRD_PAD, HEADS_PER_HALF * NRD_PAD)],
        table_v, tsem)
    tcopy.start()
    icopy = pltpu.make_async_copy(
        idx_hbm.at[pl.ds(rb * SLAB, SLAB)], idx_v, isem)
    icopy.start()
    tcopy.wait()
    icopy.wait()

    ocopies = [None, None]
    for dh in range(HEADS_PER_HALF):
        s = dh % 2
        if ocopies[s] is not None:
            ocopies[s].wait()
        out_v = out_bufs[s]
        table_h = table_v.at[pl.ds(dh * NRD_PAD, NRD_PAD)]

        @plsc.parallel_loop(0, SLAB, 16, unroll=8)
        def win_body(i, out_v=out_v, table_h=table_h):
            idx16 = idx_v[pl.ds(i, 16)]
            out_v[pl.ds(i, 16)] = plsc.load_gather(table_h, [idx16])

        ocopies[s] = pltpu.make_async_copy(
            out_v,
            pos_hbm.at[pl.ds(((h0 + dh) * NB_R + rb) * SLAB, SLAB)],
            osems[s])
        ocopies[s].start()

    for s in range(2):
        if ocopies[s] is not None:
            ocopies[s].wait()


def _sc_gather(table_t_pad, idx_pad_flat):
    mesh = plsc.VectorSubcoreMesh(core_axis_name="c", subcore_axis_name="s")
    fn = functools.partial(
        pl.kernel,
        mesh=mesh,
        out_type=jax.ShapeDtypeStruct((NUM_HEADS * NB_R * SLAB,), jnp.float32),
        scratch_types=[
            pltpu.VMEM((HEADS_PER_HALF * NRD_PAD,), jnp.float32),
            pltpu.VMEM((SLAB,), jnp.int32),
            pltpu.VMEM((SLAB,), jnp.float32),
            pltpu.VMEM((SLAB,), jnp.float32),
            pltpu.SemaphoreType.DMA,
            pltpu.SemaphoreType.DMA,
            pltpu.SemaphoreType.DMA,
            pltpu.SemaphoreType.DMA,
        ],
        compiler_params=pltpu.CompilerParams(needs_layout_passes=False),
    )(_sc_gather_body)
    return fn(table_t_pad, idx_pad_flat)


def _add_body(a_ref, p_ref, o_ref):
    o_ref[...] = a_ref[...] + p_ref[:, :SEQ, :SEQ][None]


def _tc_add(attn, pos_pad):
    b, nh, s, _ = attn.shape
    return pl.pallas_call(
        _add_body,
        grid=(nh,),
        in_specs=[
            pl.BlockSpec((b, 1, s, s), lambda h: (0, h, 0, 0)),
            pl.BlockSpec((1, SEQ + 7, SP), lambda h: (h, 0, 0)),
        ],
        out_specs=pl.BlockSpec((b, 1, s, s), lambda h: (0, h, 0, 0)),
        out_shape=jax.ShapeDtypeStruct(attn.shape, attn.dtype),
        compiler_params=pltpu.CompilerParams(
            vmem_limit_bytes=110 * 1024 * 1024,
        ),
    )(attn, pos_pad)


def kernel(attention_scores, relative_position_bias_table, relative_position_index):
    nrd = relative_position_bias_table.shape[0]
    table_t_pad = jnp.pad(
        jnp.transpose(relative_position_bias_table),
        ((0, 0), (0, NRD_PAD - nrd)),
    ).reshape(-1)
    idx_pad_flat = jnp.pad(
        relative_position_index,
        ((0, ROWS_PAD - SEQ), (0, SP - SEQ)),
    ).reshape(-1)
    pos_flat = _sc_gather(table_t_pad, idx_pad_flat)
    pos_pad = pos_flat.reshape(NUM_HEADS, ROWS_PAD, SP)
    return _tc_add(attention_scores, pos_pad)


# speedup vs baseline: 1.0765x; 1.0765x over previous
import functools

import jax
import jax.numpy as jnp
from jax import lax
from jax.experimental import pallas as pl
from jax.experimental.pallas import tpu as pltpu
from jax.experimental.pallas import tpu_sc as plsc

SEQ = 577
NUM_HEADS = 12
NB_R = 16
R_BLK = 40
ROWS_PAD = NB_R * R_BLK
SP = 584
HEAD_HALVES = 2
HEADS_PER_HALF = NUM_HEADS // HEAD_HALVES
COL_OFFS = tuple(range(0, SP - 16, 16)) + (SP - 16,)


def _sc_gather_body(nrd, table_hbm, idx_hbm, pos_hbm,
                    table_v, idx_v, out_v0, out_v1, tsem, isem, osem0, osem1):
    out_bufs = (out_v0, out_v1)
    osems = (osem0, osem1)
    wid = lax.axis_index("s") * 2 + lax.axis_index("c")
    hh = wid // NB_R
    rb = wid % NB_R
    r0 = rb * R_BLK
    h0 = hh * HEADS_PER_HALF

    tcopy = pltpu.make_async_copy(table_hbm, table_v, tsem)
    tcopy.start()
    icopy = pltpu.make_async_copy(idx_hbm.at[pl.ds(r0, R_BLK), :], idx_v, isem)
    icopy.start()
    tcopy.wait()
    icopy.wait()

    ocopies = [None, None]
    for dh in range(HEADS_PER_HALF):
        s = dh % 2
        if ocopies[s] is not None:
            ocopies[s].wait()
        out_v = out_bufs[s]
        hoff = (h0 + dh) * nrd

        @plsc.parallel_loop(0, R_BLK, unroll=8)
        def row_body(r, out_v=out_v, hoff=hoff):
            for off in COL_OFFS:
                idx16 = idx_v[r, pl.ds(off, 16)]
                out_v[r, pl.ds(off, 16)] = plsc.load_gather(
                    table_v, [idx16 + hoff])

        ocopies[s] = pltpu.make_async_copy(
            out_v, pos_hbm.at[h0 + dh, pl.ds(r0, R_BLK), :], osems[s])
        ocopies[s].start()

    for s in range(2):
        if ocopies[s] is not None:
            ocopies[s].wait()


def _sc_gather(table_t_flat, idx_pad, nrd):
    mesh = plsc.VectorSubcoreMesh(core_axis_name="c", subcore_axis_name="s")
    fn = functools.partial(
        pl.kernel,
        mesh=mesh,
        out_type=jax.ShapeDtypeStruct((NUM_HEADS, ROWS_PAD, SP), jnp.float32),
        scratch_types=[
            pltpu.VMEM((NUM_HEADS * nrd,), jnp.float32),
            pltpu.VMEM((R_BLK, SP), jnp.int32),
            pltpu.VMEM((R_BLK, SP), jnp.float32),
            pltpu.VMEM((R_BLK, SP), jnp.float32),
            pltpu.SemaphoreType.DMA,
            pltpu.SemaphoreType.DMA,
            pltpu.SemaphoreType.DMA,
            pltpu.SemaphoreType.DMA,
        ],
        compiler_params=pltpu.CompilerParams(needs_layout_passes=False),
    )(functools.partial(_sc_gather_body, nrd))
    return fn(table_t_flat, idx_pad)


def _add_body(a_ref, p_ref, o_ref):
    o_ref[...] = a_ref[...] + p_ref[:, :SEQ, :SEQ][None]


def _tc_add(attn, pos_pad):
    b, nh, s, _ = attn.shape
    return pl.pallas_call(
        _add_body,
        grid=(nh,),
        in_specs=[
            pl.BlockSpec((b, 1, s, s), lambda h: (0, h, 0, 0)),
            pl.BlockSpec((1, SP, SP), lambda h: (h, 0, 0)),
        ],
        out_specs=pl.BlockSpec((b, 1, s, s), lambda h: (0, h, 0, 0)),
        out_shape=jax.ShapeDtypeStruct(attn.shape, attn.dtype),
        compiler_params=pltpu.CompilerParams(
            vmem_limit_bytes=110 * 1024 * 1024,
        ),
    )(attn, pos_pad)


def kernel(attention_scores, relative_position_bias_table, relative_position_index):
    nrd = relative_position_bias_table.shape[0]
    table_t_flat = jnp.transpose(relative_position_bias_table).reshape(-1)
    idx_pad = jnp.pad(
        relative_position_index,
        ((0, ROWS_PAD - SEQ), (0, SP - SEQ)),
    )
    pos_pad = _sc_gather(table_t_flat, idx_pad, nrd)
    return _tc_add(attention_scores, pos_pad)
